# baseline (speedup 1.0000x reference)
import functools

import jax
import jax.numpy as jnp
from jax import lax
from jax.experimental import pallas as pl
from jax.experimental.pallas import tpu as pltpu
from jax.experimental.pallas import tpu_sc as plsc

CH = 40
VIA_A = [1, 2, 3, 5, 6, 7]
VIA_B = [9, 10, 11, 13, 14, 15]
DIR_A = [0, 4]
DIR_B = [8, 12]
BODY = 16


def _sc_gather(table, idx_flat):
    B = idx_flat.shape[0]
    D = table.shape[1]
    info = plsc.get_sparse_core_info()
    NC, NS = info.num_cores, info.num_subcores
    NW = NC * NS
    b_per_w = B // NW
    n_chunks = b_per_w // CH
    n_steps = n_chunks // BODY
    assert b_per_w % CH == 0 and n_chunks % BODY == 0

    mesh = plsc.VectorSubcoreMesh(core_axis_name="c", subcore_axis_name="s")

    @functools.partial(
        pl.kernel,
        mesh=mesh,
        out_type=jax.ShapeDtypeStruct((B, D), jnp.float32),
        scratch_types=[
            pltpu.VMEM((b_per_w,), jnp.int32),
            pltpu.VMEM_SHARED((NS, 6, CH, D), jnp.float32),
            *[pltpu.VMEM((CH, D), jnp.float32) for _ in range(10)],
            *[pltpu.SemaphoreType.DMA for _ in range(10)],
            *[pltpu.SemaphoreType.DMA for _ in range(6)],
            *[pltpu.SemaphoreType.DMA for _ in range(6)],
            *[pltpu.SemaphoreType.DMA for _ in range(4)],
        ],
    )
    def k(table_hbm, idx_hbm, out_hbm, idx_v, spm, *rest):
        v = rest[0:6]
        d = rest[6:10]
        gv = rest[10:16]
        gd = rest[16:20]
        xsem = rest[20:26]
        dsem = rest[26:32]
        wsem = rest[32:36]
        sid = lax.axis_index("s")
        wid = sid * NC + lax.axis_index("c")
        base = wid * b_per_w

        pltpu.sync_copy(idx_hbm.at[pl.ds(base, b_per_w)], idx_v)

        def gather_copy(g, buf, sem):
            return pltpu.make_async_copy(
                table_hbm.at[idx_v.at[pl.ds(g * CH, CH)]], buf, sem
            )

        def write_copy(g, buf, sem):
            return pltpu.make_async_copy(
                buf, out_hbm.at[pl.ds(base + g * CH, CH)], sem
            )

        def xbar_copy(k_, buf):
            return pltpu.make_async_copy(buf, spm.at[sid, k_], xsem[k_])

        def drain_copy(g, k_):
            return pltpu.make_async_copy(
                spm.at[sid, k_], out_hbm.at[pl.ds(base + g * CH, CH)], dsem[k_]
            )

        for k_, j in enumerate(VIA_A):
            gather_copy(j, v[k_], gv[k_]).start()
        for l_, j in enumerate(DIR_A + DIR_B):
            gather_copy(j, d[l_], gd[l_]).start()

        def body(s, carry):
            g0 = s * BODY
            for k_, j in enumerate(VIA_A):
                g = g0 + j
                gather_copy(g, v[k_], gv[k_]).wait()

                @pl.when(s > 0)
                def _():
                    drain_copy(g, k_).wait()

                xbar_copy(k_, v[k_]).start()
            for k_, j in enumerate(VIA_A):
                g = g0 + j
                xbar_copy(k_, v[k_]).wait()
                drain_copy(g, k_).start()
                gather_copy(g + 8, v[k_], gv[k_]).start()
            for l_, j in enumerate(DIR_A):
                g = g0 + j
                gather_copy(g, d[l_], gd[l_]).wait()
                write_copy(g, d[l_], wsem[l_]).start()
            for k_, j in enumerate(VIA_B):
                g = g0 + j
                gather_copy(g, v[k_], gv[k_]).wait()
                drain_copy(g, k_).wait()
                xbar_copy(k_, v[k_]).start()
            for k_, j in enumerate(VIA_B):
                g = g0 + j
                xbar_copy(k_, v[k_]).wait()
                drain_copy(g, k_).start()

                @pl.when(s < n_steps - 1)
                def _():
                    gather_copy(g0 + BODY + VIA_A[k_], v[k_], gv[k_]).start()
            for l_, j in enumerate(DIR_B):
                g = g0 + j
                gather_copy(g, d[2 + l_], gd[2 + l_]).wait()
                write_copy(g, d[2 + l_], wsem[2 + l_]).start()
            for l_, j in enumerate(DIR_A + DIR_B):
                write_copy(g0 + j, d[l_], wsem[l_]).wait()

                @pl.when(s < n_steps - 1)
                def _():
                    gather_copy(g0 + BODY + j, d[l_], gd[l_]).start()
            return carry

        lax.fori_loop(0, n_steps, body, 0)
        for k_, j in enumerate(VIA_B):
            drain_copy((n_steps - 1) * BODY + j, k_).wait()

    return k(table, idx_flat)


def kernel(x, emb):
    S0, S1 = x.shape
    idx_t = x.T.reshape(S0 * S1).astype(jnp.int32)
    out = _sc_gather(emb, idx_t)
    return out.reshape(S1, S0, emb.shape[1]).transpose(1, 0, 2)


# speedup vs baseline: 1.0111x; 1.0111x over previous
import functools

import jax
import jax.numpy as jnp
from jax import lax
from jax.experimental import pallas as pl
from jax.experimental.pallas import tpu as pltpu
from jax.experimental.pallas import tpu_sc as plsc

CH = 40
VIA_A = [1, 2, 3, 5, 6, 7]
VIA_B = [9, 10, 11, 13, 14, 15]
DIR_A = [0, 4]
DIR_B = [8, 12]
BODY = 16


def _sc_gather(table, idx_flat):
    B = idx_flat.shape[0]
    D = table.shape[1]
    info = plsc.get_sparse_core_info()
    NC, NS = info.num_cores, info.num_subcores
    NW = NC * NS
    b_per_w = B // NW
    n_chunks = b_per_w // CH
    n_steps = n_chunks // BODY
    assert b_per_w % CH == 0 and n_chunks % BODY == 0

    mesh = plsc.VectorSubcoreMesh(core_axis_name="c", subcore_axis_name="s")

    @functools.partial(
        pl.kernel,
        mesh=mesh,
        out_type=jax.ShapeDtypeStruct((B, D), jnp.float32),
        scratch_types=[
            pltpu.VMEM((b_per_w,), jnp.int32),
            pltpu.VMEM_SHARED((NS, 6, CH, D), jnp.float32),
            *[pltpu.VMEM((CH, D), jnp.float32) for _ in range(10)],
            *[pltpu.SemaphoreType.DMA for _ in range(10)],
            *[pltpu.SemaphoreType.DMA for _ in range(6)],
            *[pltpu.SemaphoreType.DMA for _ in range(6)],
            *[pltpu.SemaphoreType.DMA for _ in range(4)],
        ],
    )
    def k(table_hbm, idx_hbm, out_hbm, idx_v, spm, *rest):
        v = rest[0:6]
        d = rest[6:10]
        gv = rest[10:16]
        gd = rest[16:20]
        xsem = rest[20:26]
        dsem = rest[26:32]
        wsem = rest[32:36]
        sid = lax.axis_index("s")
        wid = sid * NC + lax.axis_index("c")
        base = wid * b_per_w

        pltpu.sync_copy(idx_hbm.at[pl.ds(base, b_per_w)], idx_v)

        def gather_copy(g, buf, sem):
            return pltpu.make_async_copy(
                table_hbm.at[idx_v.at[pl.ds(g * CH, CH)]], buf, sem
            )

        def write_copy(g, buf, sem):
            return pltpu.make_async_copy(
                buf, out_hbm.at[pl.ds(base + g * CH, CH)], sem
            )

        def xbar_copy(k_, buf):
            return pltpu.make_async_copy(buf, spm.at[sid, k_], xsem[k_])

        def drain_copy(g, k_):
            return pltpu.make_async_copy(
                spm.at[sid, k_], out_hbm.at[pl.ds(base + g * CH, CH)], dsem[k_]
            )

        for k_, j in enumerate(VIA_A):
            gather_copy(j, v[k_], gv[k_]).start()
        for l_, j in enumerate(DIR_A + DIR_B):
            gather_copy(j, d[l_], gd[l_]).start()

        def body(s, carry):
            g0 = s * BODY
            for k_, j in enumerate(VIA_A):
                g = g0 + j
                gather_copy(g, v[k_], gv[k_]).wait()

                @pl.when(s > 0)
                def _():
                    drain_copy(g, k_).wait()

                xbar_copy(k_, v[k_]).start()
                xbar_copy(k_, v[k_]).wait()
                drain_copy(g, k_).start()
                gather_copy(g + 8, v[k_], gv[k_]).start()
            for l_, j in enumerate(DIR_A):
                g = g0 + j
                gather_copy(g, d[l_], gd[l_]).wait()
                write_copy(g, d[l_], wsem[l_]).start()
            for k_, j in enumerate(VIA_B):
                g = g0 + j
                gather_copy(g, v[k_], gv[k_]).wait()
                drain_copy(g, k_).wait()
                xbar_copy(k_, v[k_]).start()
                xbar_copy(k_, v[k_]).wait()
                drain_copy(g, k_).start()

                @pl.when(s < n_steps - 1)
                def _():
                    gather_copy(g0 + BODY + VIA_A[k_], v[k_], gv[k_]).start()
            for l_, j in enumerate(DIR_B):
                g = g0 + j
                gather_copy(g, d[2 + l_], gd[2 + l_]).wait()
                write_copy(g, d[2 + l_], wsem[2 + l_]).start()
            for l_, j in enumerate(DIR_A + DIR_B):
                write_copy(g0 + j, d[l_], wsem[l_]).wait()

                @pl.when(s < n_steps - 1)
                def _():
                    gather_copy(g0 + BODY + j, d[l_], gd[l_]).start()
            return carry

        lax.fori_loop(0, n_steps, body, 0)
        for k_, j in enumerate(VIA_B):
            drain_copy((n_steps - 1) * BODY + j, k_).wait()

    return k(table, idx_flat)


def kernel(x, emb):
    S0, S1 = x.shape
    idx_t = x.T.reshape(S0 * S1).astype(jnp.int32)
    out = _sc_gather(emb, idx_t)
    return out.reshape(S1, S0, emb.shape[1]).transpose(1, 0, 2)
